# Initial kernel scaffold; baseline (speedup 1.0000x reference)
import jax
import jax.numpy as jnp
from jax.experimental import pallas as pl


def kernel(nodes_feats, edge_index, edge_attr, agent_id, entity_embed, W1, b1, ln_g, ln_b, Wl1, bl1, Wl2, bl2, Wq1, bq1, Wk1, bk1, Wv1, bv1, We1, Ws1, bs1, Wq2, bq2, Wk2, bk2, Wv2, bv2, We2, Ws2, bs2):
    raise NotImplementedError("write your pallas kernel here")


# speedup vs baseline: 1.0543x; 1.0543x over previous
import functools

import jax
import jax.numpy as jnp
from jax.experimental import pallas as pl
from jax.experimental.pallas import tpu as pltpu

B_ = 50
N_ = 1000
E_ = 32000
NN = B_ * N_
EE = B_ * E_
HID = 16
OUT = 16
HEADS = 2

EBLK = 12800
NEB = EE // EBLK


def _ln(x, g, b):
    mu = jnp.mean(x, axis=-1, keepdims=True)
    var = jnp.mean((x - mu) ** 2, axis=-1, keepdims=True)
    return (x - mu) * jax.lax.rsqrt(var + 1e-5) * g + b


def _mm_T(W, x):
    O, K = W.shape
    Wr = W.astype(jnp.bfloat16).astype(jnp.float32)
    xr = x.astype(jnp.bfloat16).astype(jnp.float32)
    acc = Wr[:, 0:1] * xr[0:1, :]
    for j in range(1, K):
        acc = acc + Wr[:, j:j + 1] * xr[j:j + 1, :]
    return acc


def _ln_T(x, g, b):
    mu = jnp.mean(x, axis=0, keepdims=True)
    var = jnp.mean((x - mu) ** 2, axis=0, keepdims=True)
    return (x - mu) * jax.lax.rsqrt(var + 1e-5) * g + b


def _edge_mlp_body(ent_ref, ea_ref, TT_ref, W1b_ref, b1_ref, g_ref, bb_ref,
                   Wl1_ref, bl1_ref, Wl2_ref, bl2_ref, h_ref):
    ent = ent_ref[...]
    ea = ea_ref[...]
    g = g_ref[...]
    bb = bb_ref[...]
    TT = TT_ref[...]
    p1 = jnp.zeros((16, EBLK), jnp.float32)
    for j in range(5):
        p1 = jnp.where(ent == j, TT[:, j:j + 1], p1)
    p2 = _mm_T(W1b_ref[...], ea)
    h = jax.nn.relu(p1 + p2 + b1_ref[...])
    h = _ln_T(h, g, bb)
    h = jax.nn.relu(_mm_T(Wl1_ref[...], h) + bl1_ref[...])
    h = _ln_T(h, g, bb)
    h = jax.nn.relu(_mm_T(Wl2_ref[...], h) + bl2_ref[...])
    h = _ln_T(h, g, bb)
    h_ref[...] = h


def _edge_mlp(entT, eaT, T, W1b, b1, g, bb, Wl1, bl1, Wl2, bl2):
    grid = (NEB,)
    const = lambda shape: pl.BlockSpec(shape, lambda i: tuple(0 for _ in shape))
    return pl.pallas_call(
        _edge_mlp_body,
        grid=grid,
        in_specs=[
            pl.BlockSpec((1, EBLK), lambda i: (0, i)),
            pl.BlockSpec((4, EBLK), lambda i: (0, i)),
            const((16, 5)),
            const((16, 4)),
            const((16, 1)),
            const((16, 1)),
            const((16, 1)),
            const((16, 16)),
            const((16, 1)),
            const((16, 16)),
            const((16, 1)),
        ],
        out_specs=pl.BlockSpec((16, EBLK), lambda i: (0, i)),
        out_shape=jax.ShapeDtypeStruct((16, EE), jnp.float32),
    )(entT, eaT, T.T, W1b, b1.reshape(16, 1), g.reshape(16, 1), bb.reshape(16, 1),
      Wl1, bl1.reshape(16, 1), Wl2, bl2.reshape(16, 1))


def _tconv(x, src, dst, ea, Wq, bq, Wk, bk, Wv, bv, We, Ws, bs, heads, outc):
    q = (x @ Wq.T + bq)[dst].reshape(-1, heads, outc)
    k = (x @ Wk.T + bk)[src].reshape(-1, heads, outc)
    v = (x @ Wv.T + bv)[src].reshape(-1, heads, outc)
    e = (ea @ We.T).reshape(-1, heads, outc)
    k = k + e
    alpha = jnp.sum(q * k, axis=-1) / jnp.sqrt(float(outc))
    m = jax.ops.segment_max(alpha, dst, num_segments=NN)
    m = jnp.where(jnp.isfinite(m), m, 0.0)
    ex = jnp.exp(alpha - m[dst])
    wv = (v + e) * ex[:, :, None]
    num = jax.ops.segment_sum(wv, dst, num_segments=NN)
    den = jax.ops.segment_sum(ex, dst, num_segments=NN)
    out = num / (den[:, :, None] + 1e-16)
    out = jnp.mean(out, axis=1)
    return out + x @ Ws.T + bs


def kernel(nodes_feats, edge_index, edge_attr, agent_id, entity_embed, W1, b1,
           ln_g, ln_b, Wl1, bl1, Wl2, bl2, Wq1, bq1, Wk1, bk1, Wv1, bv1, We1,
           Ws1, bs1, Wq2, bq2, Wk2, bk2, Wv2, bv2, We2, Ws2, bs2):
    adder = N_ * jnp.arange(B_, dtype=jnp.float32)[:, None, None]
    ei = jnp.transpose(edge_index + adder, (1, 0, 2)).reshape(2, -1).astype(jnp.int32)
    src, dst = ei[0], ei[1]
    node_ent = nodes_feats.reshape(-1).astype(jnp.int32)
    ea = edge_attr.reshape(EE, -1)
    ent = node_ent[src]

    T = entity_embed @ W1[:, :4].T
    hT = _edge_mlp(ent.reshape(1, EE), ea.T, T, W1[:, 4:], b1, ln_g, ln_b,
                   Wl1, bl1, Wl2, bl2)
    h = hT.T

    nodes = jax.ops.segment_sum(h, dst, num_segments=NN)
    A = _tconv(nodes, src, dst, ea, Wq1, bq1, Wk1, bk1, Wv1, bv1, We1, Ws1, bs1, HEADS, HID)
    A = _tconv(A, src, dst, ea, Wq2, bq2, Wk2, bk2, Wv2, bv2, We2, Ws2, bs2, HEADS, OUT)
    A = jax.nn.relu(A).reshape(B_, N_, -1)
    aid = agent_id.astype(jnp.int32)[:, 0]
    return A[jnp.arange(B_), aid]


# speedup vs baseline: 13.4588x; 12.7652x over previous
import functools

import jax
import jax.numpy as jnp
from jax import lax
from jax.experimental import pallas as pl
from jax.experimental.pallas import tpu as pltpu
from jax.experimental.pallas import tpu_sc as plsc

B_ = 50
N_ = 1000
E_ = 32000
NN = B_ * N_
EE = B_ * E_
HID = 16
OUT = 16
HEADS = 2

EBLK = 12800
NEB = EE // EBLK


def _ln(x, g, b):
    mu = jnp.mean(x, axis=-1, keepdims=True)
    var = jnp.mean((x - mu) ** 2, axis=-1, keepdims=True)
    return (x - mu) * jax.lax.rsqrt(var + 1e-5) * g + b


def _mm_T(W, x):
    O, K = W.shape
    Wr = W.astype(jnp.bfloat16).astype(jnp.float32)
    xr = x.astype(jnp.bfloat16).astype(jnp.float32)
    acc = Wr[:, 0:1] * xr[0:1, :]
    for j in range(1, K):
        acc = acc + Wr[:, j:j + 1] * xr[j:j + 1, :]
    return acc


def _ln_T(x, g, b):
    mu = jnp.mean(x, axis=0, keepdims=True)
    var = jnp.mean((x - mu) ** 2, axis=0, keepdims=True)
    return (x - mu) * jax.lax.rsqrt(var + 1e-5) * g + b


def _edge_mlp_body(ent_ref, ea_ref, TT_ref, W1b_ref, b1_ref, g_ref, bb_ref,
                   Wl1_ref, bl1_ref, Wl2_ref, bl2_ref, h_ref):
    ent = ent_ref[...]
    ea = ea_ref[...]
    g = g_ref[...]
    bb = bb_ref[...]
    TT = TT_ref[...]
    p1 = jnp.zeros((16, EBLK), jnp.float32)
    for j in range(5):
        p1 = jnp.where(ent == j, TT[:, j:j + 1], p1)
    p2 = _mm_T(W1b_ref[...], ea)
    h = jax.nn.relu(p1 + p2 + b1_ref[...])
    h = _ln_T(h, g, bb)
    h = jax.nn.relu(_mm_T(Wl1_ref[...], h) + bl1_ref[...])
    h = _ln_T(h, g, bb)
    h = jax.nn.relu(_mm_T(Wl2_ref[...], h) + bl2_ref[...])
    h = _ln_T(h, g, bb)
    h_ref[...] = h


def _edge_mlp(entT, eaT, T, W1b, b1, g, bb, Wl1, bl1, Wl2, bl2):
    grid = (NEB,)
    const = lambda shape: pl.BlockSpec(shape, lambda i: tuple(0 for _ in shape))
    return pl.pallas_call(
        _edge_mlp_body,
        grid=grid,
        in_specs=[
            pl.BlockSpec((1, EBLK), lambda i: (0, i)),
            pl.BlockSpec((4, EBLK), lambda i: (0, i)),
            const((16, 5)),
            const((16, 4)),
            const((16, 1)),
            const((16, 1)),
            const((16, 1)),
            const((16, 16)),
            const((16, 1)),
            const((16, 16)),
            const((16, 1)),
        ],
        out_specs=pl.BlockSpec((16, EBLK), lambda i: (0, i)),
        out_shape=jax.ShapeDtypeStruct((16, EE), jnp.float32),
    )(entT, eaT, T.T, W1b, b1.reshape(16, 1), g.reshape(16, 1), bb.reshape(16, 1),
      Wl1, bl1.reshape(16, 1), Wl2, bl2.reshape(16, 1))


NW = 32
CH = 2048
NCHUNK = 25
EPAD = NW * CH * NCHUNK
RPS = NN // 16


NPAD = 50048
RPSUB = NPAD // 16


def _sc_segsum16(h_pad, dst2):
    mesh = plsc.VectorSubcoreMesh(core_axis_name="c", subcore_axis_name="s")

    @functools.partial(
        pl.kernel, mesh=mesh,
        out_type=jax.ShapeDtypeStruct((2, NPAD, 16), jnp.float32),
        compiler_params=pltpu.CompilerParams(use_tc_tiling_on_sc=False),
        scratch_types=[
            pltpu.VMEM((16, 128), jnp.int32),
            pltpu.VMEM((CH, 16), jnp.float32),
            pltpu.VMEM((136, 16), jnp.float32),
            pltpu.VMEM_SHARED((NPAD, 16), jnp.float32),
        ],
    )
    def k(h_hbm, d_hbm, out_hbm, idx_v, val_v, z_v, table):
        c = lax.axis_index("c")
        s = lax.axis_index("s")
        wid = s * 2 + c
        r0 = pl.multiple_of(s * RPSUB, 8)

        def zrow(i, carry):
            z_v[i, :] = jnp.zeros((16,), jnp.float32)
            return carry
        lax.fori_loop(0, 136, zrow, 0)

        def zcp(j, carry):
            pltpu.sync_copy(z_v, table.at[pl.ds(pl.multiple_of(r0 + j * 136, 8), 136)])
            return carry
        lax.fori_loop(0, 23, zcp, 0)
        plsc.subcore_barrier()

        def chunk(i, carry):
            e0 = pl.multiple_of((wid * NCHUNK + i) * CH, 2048)
            pltpu.sync_copy(d_hbm.at[pl.ds(pl.multiple_of(e0 // 128, 16), 16)], idx_v)
            pltpu.sync_copy(h_hbm.at[pl.ds(e0, CH)], val_v)
            for j in range(16):
                pltpu.sync_copy(val_v.at[pl.ds(j * 128, 128)],
                                table.at[idx_v.at[j]], add=True)
            return carry
        lax.fori_loop(0, NCHUNK, chunk, 0)
        plsc.subcore_barrier()
        pltpu.sync_copy(table.at[pl.ds(r0, RPSUB)], out_hbm.at[c, pl.ds(r0, RPSUB)])

    return k(h_pad, dst2)


def _rnd(x):
    return x.astype(jnp.bfloat16).astype(jnp.float32)


def _proj_math(x, Wq_r, bq_r, Wk_r, bk_r, Wv_r, bv_r, We_r, Ws_r, bs_r):
    xr = _rnd(x)
    dot = lambda a, b: jnp.dot(a, b, preferred_element_type=jnp.float32)
    q = dot(xr, Wq_r[...].T) + bq_r[...]
    k = dot(xr, Wk_r[...].T) + bk_r[...]
    v = dot(xr, Wv_r[...].T) + bv_r[...]
    skip = dot(xr, Ws_r[...].T) + bs_r[...]
    M = x.shape[0]
    G = dot(q, We_r[...])
    qG = jnp.concatenate([q, G, jnp.zeros((M, 8), jnp.float32)], axis=1)
    return qG, k, v[:, :16], v[:, 16:], skip


def _proj_body(p0_ref, p1_ref, Wq_r, bq_r, Wk_r, bk_r, Wv_r, bv_r, We_r, Ws_r,
               bs_r, qG_ref, k_ref, v0_ref, v1_ref, skip_ref):
    x = p0_ref[...] + p1_ref[...]
    qG, k, v0, v1, skip = _proj_math(x, Wq_r, bq_r, Wk_r, bk_r, Wv_r, bv_r,
                                     We_r, Ws_r, bs_r)
    qG_ref[...] = qG
    k_ref[...] = k
    v0_ref[...] = v0
    v1_ref[...] = v1
    skip_ref[...] = skip


NBLK = 3128


def _proj_tables(p0, p1, Wq, bq, Wk, bk, Wv, bv, We, Ws, bs):
    const = lambda shape: pl.BlockSpec(shape, lambda i: tuple(0 for _ in shape))
    row = lambda w: pl.BlockSpec((NBLK, w), lambda i: (i, 0))
    outs = pl.pallas_call(
        _proj_body,
        grid=(NPAD // NBLK,),
        in_specs=[row(16), row(16), const((32, 16)), const((1, 32)),
                  const((32, 16)), const((1, 32)), const((32, 16)), const((1, 32)),
                  const((32, 8)), const((16, 16)), const((1, 16))],
        out_specs=[row(48), row(32), row(16), row(16), row(16)],
        out_shape=[jax.ShapeDtypeStruct((NPAD, 48), jnp.float32),
                   jax.ShapeDtypeStruct((NPAD, 32), jnp.float32),
                   jax.ShapeDtypeStruct((NPAD, 16), jnp.float32),
                   jax.ShapeDtypeStruct((NPAD, 16), jnp.float32),
                   jax.ShapeDtypeStruct((NPAD, 16), jnp.float32)],
    )(p0, p1, _rnd(Wq), bq.reshape(1, 32), _rnd(Wk), bk.reshape(1, 32),
      _rnd(Wv), bv.reshape(1, 32), _weblk(We), _rnd(Ws), bs.reshape(1, 16))
    return outs


def _weblk(We):
    Wr = _rnd(We)
    z = jnp.zeros((16, 4), jnp.float32)
    return jnp.concatenate([jnp.concatenate([Wr[:16], z], axis=1),
                            jnp.concatenate([z, Wr[16:]], axis=1)], axis=0)


ECPW = EPAD // NW
ACH = 1024
NACH = ECPW // ACH


def _sc_alpha(qG_tab, k_tab, srcP, dstP, eaR):
    mesh = plsc.VectorSubcoreMesh(core_axis_name="c", subcore_axis_name="s")

    @functools.partial(
        pl.kernel, mesh=mesh,
        out_type=jax.ShapeDtypeStruct((2, EPAD), jnp.float32),
        compiler_params=pltpu.CompilerParams(use_tc_tiling_on_sc=False,
                                             needs_layout_passes=False),
        scratch_types=[
            pltpu.VMEM((ACH,), jnp.int32),
            pltpu.VMEM((ACH,), jnp.int32),
            pltpu.VMEM((ACH, 4), jnp.float32),
            pltpu.VMEM((128, 48), jnp.float32),
            pltpu.VMEM((128, 32), jnp.float32),
            pltpu.VMEM((2, ACH), jnp.float32),
        ],
    )
    def kfn(qG_hbm, k_hbm, src_hbm, dst_hbm, ea_hbm, out_hbm, dst_v, src_v,
            ea_v, qg_v, kk_v, a_v):
        c = lax.axis_index("c")
        s = lax.axis_index("s")
        wid = s * 2 + c
        lanes = jnp.arange(16, dtype=jnp.int32)

        def chunk(i, carry):
            e0 = pl.multiple_of((wid * NACH + i) * ACH, 1024)
            pltpu.sync_copy(dst_hbm.at[pl.ds(e0, ACH)], dst_v)
            pltpu.sync_copy(src_hbm.at[pl.ds(e0, ACH)], src_v)
            pltpu.sync_copy(ea_hbm.at[pl.ds(e0, ACH)], ea_v)

            def blk(j, carry2):
                b0 = j * 128
                pltpu.sync_copy(qG_hbm.at[dst_v.at[pl.ds(b0, 128)]], qg_v)
                pltpu.sync_copy(k_hbm.at[src_v.at[pl.ds(b0, 128)]], kk_v)

                def grp(g, carry3):
                    rows = g * 16 + lanes
                    erows = b0 + rows
                    eac = [plsc.load_gather(ea_v, [erows, jnp.full((16,), j4, jnp.int32)])
                           for j4 in range(4)]
                    for h in range(2):
                        acc = jnp.zeros((16,), jnp.float32)
                        for cc in range(16):
                            col = jnp.full((16,), h * 16 + cc, jnp.int32)
                            qc = plsc.load_gather(qg_v, [rows, col])
                            kc = plsc.load_gather(kk_v, [rows, col])
                            acc = acc + qc * kc
                        for j4 in range(4):
                            gcol = jnp.full((16,), 32 + h * 4 + j4, jnp.int32)
                            Gj = plsc.load_gather(qg_v, [rows, gcol])
                            acc = acc + eac[j4] * Gj
                        a_v[h, pl.ds(b0 + g * 16, 16)] = acc * 0.25
                    return carry3
                lax.fori_loop(0, 8, grp, 0)
                return carry2
            lax.fori_loop(0, ACH // 128, blk, 0)
            pltpu.sync_copy(a_v, out_hbm.at[:, pl.ds(e0, ACH)])
            return carry
        lax.fori_loop(0, NACH, chunk, 0)

    return kfn(qG_tab, k_tab, srcP, dstP, eaR)
    q = (x @ Wq.T + bq)[dst].reshape(-1, heads, outc)
    k = (x @ Wk.T + bk)[src].reshape(-1, heads, outc)
    v = (x @ Wv.T + bv)[src].reshape(-1, heads, outc)
    e = (ea @ We.T).reshape(-1, heads, outc)
    k = k + e
    alpha = jnp.sum(q * k, axis=-1) / jnp.sqrt(float(outc))
    m = jax.ops.segment_max(alpha, dst, num_segments=NN)
    m = jnp.where(jnp.isfinite(m), m, 0.0)
    ex = jnp.exp(alpha - m[dst])
    wv = (v + e) * ex[:, :, None]
    num = jax.ops.segment_sum(wv, dst, num_segments=NN)
    den = jax.ops.segment_sum(ex, dst, num_segments=NN)
    out = num / (den[:, :, None] + 1e-16)
    out = jnp.mean(out, axis=1)
    return out + x @ Ws.T + bs


def _vgather(x, idx):
    dn = lax.GatherDimensionNumbers(offset_dims=(), collapsed_slice_dims=(0,),
                                    start_index_map=(0,))
    return lax.gather(x, idx[:, None], dn, (1,),
                      mode=lax.GatherScatterMode.PROMISE_IN_BOUNDS)


def _sc_segmax(alphaT, dstP):
    mesh = plsc.VectorSubcoreMesh(core_axis_name="c", subcore_axis_name="s")

    @functools.partial(
        pl.kernel, mesh=mesh,
        out_type=jax.ShapeDtypeStruct((NW, NPAD * 2), jnp.float32),
        compiler_params=pltpu.CompilerParams(use_tc_tiling_on_sc=False,
                                             needs_layout_passes=False),
        scratch_types=[
            pltpu.VMEM((ACH,), jnp.int32),
            pltpu.VMEM((2, ACH), jnp.float32),
            pltpu.VMEM((NPAD * 2,), jnp.float32),
        ],
    )
    def kfn(a_hbm, dst_hbm, out_hbm, dst_v, a_v, tab):
        c = lax.axis_index("c")
        s = lax.axis_index("s")
        wid = s * 2 + c
        lanes = jnp.arange(16, dtype=jnp.int32)
        ninf = jnp.full((16,), -jnp.inf, jnp.float32)

        def initr(i, carry):
            tab[pl.ds(i * 16, 16)] = ninf
            return carry
        lax.fori_loop(0, NPAD * 2 // 16, initr, 0)

        def chunk(i, carry):
            e0 = pl.multiple_of((wid * NACH + i) * ACH, 1024)
            pltpu.sync_copy(dst_hbm.at[pl.ds(e0, ACH)], dst_v)
            pltpu.sync_copy(a_hbm.at[:, pl.ds(e0, ACH)], a_v)

            def grp(g, carry2):
                d16 = dst_v[pl.ds(g * 16, 16)]
                a0 = a_v[0, pl.ds(g * 16, 16)]
                a1 = a_v[1, pl.ds(g * 16, 16)]
                sd, sa0 = plsc.sort_key_val(d16, a0)
                _, sa1 = plsc.sort_key_val(d16, a1)
                for sh in (1, 2, 4, 8):
                    idx = jnp.maximum(lanes - sh, 0)
                    dsh = _vgather(sd, idx)
                    ok = (dsh == sd) & (lanes >= sh)
                    sa0 = jnp.where(ok, jnp.maximum(sa0, _vgather(sa0, idx)), sa0)
                    sa1 = jnp.where(ok, jnp.maximum(sa1, _vgather(sa1, idx)), sa1)
                nd = _vgather(sd, jnp.minimum(lanes + 1, 15))
                mend = (sd != nd) | (lanes == 15)
                f0 = sd * 2
                cur0 = plsc.load_gather(tab, [f0], mask=mend)
                plsc.store_scatter(tab, [f0], jnp.maximum(cur0, sa0), mask=mend)
                f1 = f0 + 1
                cur1 = plsc.load_gather(tab, [f1], mask=mend)
                plsc.store_scatter(tab, [f1], jnp.maximum(cur1, sa1), mask=mend)
                return carry2
            lax.fori_loop(0, ACH // 16, grp, 0)
            return carry
        lax.fori_loop(0, NACH, chunk, 0)
        pltpu.sync_copy(tab, out_hbm.at[wid])

    return kfn(alphaT, dstP)


def _mfix_body(p_ref, m_ref):
    x = p_ref[...]
    m = jnp.max(x, axis=0, keepdims=True)
    m_ref[...] = jnp.where(jnp.isfinite(m), m, 0.0)


def _segmax_combine(parts):
    m = pl.pallas_call(
        _mfix_body,
        in_specs=[pl.BlockSpec((NW, NPAD * 2), lambda: (0, 0))],
        out_specs=pl.BlockSpec((1, NPAD * 2), lambda: (0, 0)),
        out_shape=jax.ShapeDtypeStruct((1, NPAD * 2), jnp.float32),
    )(parts)
    return m.reshape(NPAD, 2)


def _sc_wv(v_tab, m_tab, alphaT, srcP, dst2, eaR, h):
    mesh = plsc.VectorSubcoreMesh(core_axis_name="c", subcore_axis_name="s")

    @functools.partial(
        pl.kernel, mesh=mesh,
        out_type=jax.ShapeDtypeStruct((2, NPAD, 24), jnp.float32),
        compiler_params=pltpu.CompilerParams(use_tc_tiling_on_sc=False,
                                             needs_layout_passes=False),
        scratch_types=[
            pltpu.VMEM((ACH,), jnp.int32),
            pltpu.VMEM((8, 128), jnp.int32),
            pltpu.VMEM((ACH, 4), jnp.float32),
            pltpu.VMEM((ACH,), jnp.float32),
            pltpu.VMEM((128, 16), jnp.float32),
            pltpu.VMEM((128, 2), jnp.float32),
            pltpu.VMEM((128, 24), jnp.float32),
            pltpu.VMEM((136, 24), jnp.float32),
            pltpu.VMEM_SHARED((NPAD, 24), jnp.float32),
        ],
    )
    def kfn(v_hbm, m_hbm, a_hbm, src_hbm, d2_hbm, ea_hbm, out_hbm, src_v,
            dstr, ea_v, a_v, vb, mb, stage, zb, table):
        cc = lax.axis_index("c")
        s = lax.axis_index("s")
        wid = s * 2 + cc
        r0 = pl.multiple_of(s * RPSUB, 8)
        lanes = jnp.arange(16, dtype=jnp.int32)
        z16 = jnp.zeros((16,), jnp.float32)

        def zrow(i, carry):
            zb[i, pl.ds(0, 16)] = z16
            zb[i, pl.ds(8, 16)] = z16
            return carry
        lax.fori_loop(0, 136, zrow, 0)

        def srow(i, carry):
            stage[i, pl.ds(8, 16)] = z16
            return carry
        lax.fori_loop(0, 128, srow, 0)

        def zcp(j, carry):
            pltpu.sync_copy(zb, table.at[pl.ds(pl.multiple_of(r0 + j * 136, 8), 136)])
            return carry
        lax.fori_loop(0, 23, zcp, 0)
        plsc.subcore_barrier()

        def chunk(i, carry):
            e0 = pl.multiple_of((wid * NACH + i) * ACH, 1024)
            pltpu.sync_copy(src_hbm.at[pl.ds(e0, ACH)], src_v)
            pltpu.sync_copy(d2_hbm.at[pl.ds(pl.multiple_of(e0 // 128, 8), 8)], dstr)
            pltpu.sync_copy(ea_hbm.at[pl.ds(e0, ACH)], ea_v)
            pltpu.sync_copy(a_hbm.at[h, pl.ds(e0, ACH)], a_v)

            def blk(j, carry2):
                pltpu.sync_copy(v_hbm.at[src_v.at[pl.ds(j * 128, 128)]], vb)
                pltpu.sync_copy(m_hbm.at[dstr.at[j]], mb)

                def grp(g, carry3):
                    rows = g * 16 + lanes
                    mh = plsc.load_gather(mb, [rows, jnp.full((16,), h, jnp.int32)])
                    av = a_v[pl.ds(j * 128 + g * 16, 16)]
                    ex = jnp.exp(av - mh)
                    for col in range(16):
                        fc = jnp.full((16,), col, jnp.int32)
                        vc = plsc.load_gather(vb, [rows, fc])
                        plsc.store_scatter(stage, [rows, fc], vc * ex)
                    for j4 in range(4):
                        fj = jnp.full((16,), j4, jnp.int32)
                        eac = plsc.load_gather(ea_v, [j * 128 + rows, fj])
                        plsc.store_scatter(stage, [rows, jnp.full((16,), 16 + j4, jnp.int32)], eac * ex)
                    plsc.store_scatter(stage, [rows, jnp.full((16,), 20, jnp.int32)], ex)
                    return carry3
                lax.fori_loop(0, 8, grp, 0)
                pltpu.sync_copy(stage, table.at[dstr.at[j]], add=True)
                return carry2
            lax.fori_loop(0, ACH // 128, blk, 0)
            return carry
        lax.fori_loop(0, NACH, chunk, 0)
        plsc.subcore_barrier()
        pltpu.sync_copy(table.at[pl.ds(r0, RPSUB)], out_hbm.at[cc, pl.ds(r0, RPSUB)])

    return kfn(v_tab, m_tab, alphaT, srcP, dst2, eaR)


def _node_math(d0, d1, skip, WeT):
    s0 = d0[0] + d0[1]
    s1 = d1[0] + d1[1]
    outs = []
    for hh, sh in enumerate((s0, s1)):
        r = sh[:, 16:20]
        nume = r[:, 0:1] * WeT[4 * hh:4 * hh + 1, 16 * hh:16 * hh + 16]
        for j in range(1, 4):
            nume = nume + r[:, j:j + 1] * WeT[4 * hh + j:4 * hh + j + 1,
                                              16 * hh:16 * hh + 16]
        num = sh[:, :16] + nume
        den = sh[:, 20:21]
        outs.append(num / (den + 1e-16))
    return 0.5 * (outs[0] + outs[1]) + skip


def _nodeout_proj(d0, d1, skip, WeBlk_cur, Wq, bq, Wk, bk, Wv, bv, We, Ws, bs):
    const = lambda shape: pl.BlockSpec(shape, lambda i: tuple(0 for _ in shape))
    row = lambda w: pl.BlockSpec((NBLK, w), lambda i: (i, 0))
    d_spec = pl.BlockSpec((2, NBLK, 24), lambda i: (0, i, 0))

    def body(d0_ref, d1_ref, skip_ref, WeC_ref, Wq_r, bq_r, Wk_r, bk_r, Wv_r,
             bv_r, We_r, Ws_r, bs_r, qG_ref, k_ref, v0_ref, v1_ref, skip2_ref):
        A = _node_math(d0_ref[...], d1_ref[...], skip_ref[...], WeC_ref[...])
        qG, k, v0, v1, sk = _proj_math(A, Wq_r, bq_r, Wk_r, bk_r, Wv_r, bv_r,
                                       We_r, Ws_r, bs_r)
        qG_ref[...] = qG
        k_ref[...] = k
        v0_ref[...] = v0
        v1_ref[...] = v1
        skip2_ref[...] = sk

    return pl.pallas_call(
        body,
        grid=(NPAD // NBLK,),
        in_specs=[d_spec, d_spec, row(16), const((8, 32)),
                  const((32, 16)), const((1, 32)), const((32, 16)), const((1, 32)),
                  const((32, 16)), const((1, 32)), const((32, 8)), const((16, 16)),
                  const((1, 16))],
        out_specs=[row(48), row(32), row(16), row(16), row(16)],
        out_shape=[jax.ShapeDtypeStruct((NPAD, 48), jnp.float32),
                   jax.ShapeDtypeStruct((NPAD, 32), jnp.float32),
                   jax.ShapeDtypeStruct((NPAD, 16), jnp.float32),
                   jax.ShapeDtypeStruct((NPAD, 16), jnp.float32),
                   jax.ShapeDtypeStruct((NPAD, 16), jnp.float32)],
    )(d0, d1, skip, WeBlk_cur, _rnd(Wq), bq.reshape(1, 32), _rnd(Wk),
      bk.reshape(1, 32), _rnd(Wv), bv.reshape(1, 32), _weblk(We), _rnd(Ws),
      bs.reshape(1, 16))


def _nodeout_final(d0, d1, skip, WeBlk_cur):
    row = lambda w: pl.BlockSpec((NBLK, w), lambda i: (i, 0))
    const = lambda shape: pl.BlockSpec(shape, lambda i: tuple(0 for _ in shape))
    d_spec = pl.BlockSpec((2, NBLK, 24), lambda i: (0, i, 0))

    def body(d0_ref, d1_ref, skip_ref, WeC_ref, A_ref):
        A = _node_math(d0_ref[...], d1_ref[...], skip_ref[...], WeC_ref[...])
        A_ref[...] = jax.nn.relu(A)

    return pl.pallas_call(
        body,
        grid=(NPAD // NBLK,),
        in_specs=[d_spec, d_spec, row(16), const((8, 32))],
        out_specs=row(16),
        out_shape=jax.ShapeDtypeStruct((NPAD, 16), jnp.float32),
    )(d0, d1, skip, WeBlk_cur)


def _tconv(x, src, dst, ea, Wq, bq, Wk, bk, Wv, bv, We, Ws, bs, heads, outc):
    q = (x @ Wq.T + bq)[dst].reshape(-1, heads, outc)
    k = (x @ Wk.T + bk)[src].reshape(-1, heads, outc)
    v = (x @ Wv.T + bv)[src].reshape(-1, heads, outc)
    e = (ea @ We.T).reshape(-1, heads, outc)
    k = k + e
    alpha = jnp.sum(q * k, axis=-1) / jnp.sqrt(float(outc))
    m = jax.ops.segment_max(alpha, dst, num_segments=NN)
    m = jnp.where(jnp.isfinite(m), m, 0.0)
    ex = jnp.exp(alpha - m[dst])
    wv = (v + e) * ex[:, :, None]
    num = jax.ops.segment_sum(wv, dst, num_segments=NN)
    den = jax.ops.segment_sum(ex, dst, num_segments=NN)
    out = num / (den[:, :, None] + 1e-16)
    out = jnp.mean(out, axis=1)
    return out + x @ Ws.T + bs


def kernel(nodes_feats, edge_index, edge_attr, agent_id, entity_embed, W1, b1,
           ln_g, ln_b, Wl1, bl1, Wl2, bl2, Wq1, bq1, Wk1, bk1, Wv1, bv1, We1,
           Ws1, bs1, Wq2, bq2, Wk2, bk2, Wv2, bv2, We2, Ws2, bs2):
    adder = N_ * jnp.arange(B_, dtype=jnp.float32)[:, None, None]
    ei = jnp.transpose(edge_index + adder, (1, 0, 2)).reshape(2, -1).astype(jnp.int32)
    src, dst = ei[0], ei[1]
    node_ent = nodes_feats.reshape(-1).astype(jnp.int32)
    ea = edge_attr.reshape(EE, -1)
    ent = node_ent[src]

    T = entity_embed @ W1[:, :4].T
    hT = _edge_mlp(ent.reshape(1, EE), ea.T, T, W1[:, 4:], b1, ln_g, ln_b,
                   Wl1, bl1, Wl2, bl2)
    h = hT.T

    h_pad = jnp.concatenate([h, jnp.zeros((EPAD - EE, 16), jnp.float32)], axis=0)
    dst_pad = jnp.concatenate([dst, jnp.full((EPAD - EE,), NN, jnp.int32)])
    src_pad = jnp.concatenate([src, jnp.zeros((EPAD - EE,), jnp.int32)])
    dst2 = dst_pad.reshape(EPAD // 128, 128)
    eaR = jnp.concatenate([_rnd(ea), jnp.zeros((EPAD - EE, 4), jnp.float32)], axis=0)
    parts = _sc_segsum16(h_pad, dst2)

    qG1, k1, v01, v11, skip1 = _proj_tables(parts[0], parts[1], Wq1, bq1, Wk1,
                                            bk1, Wv1, bv1, We1, Ws1, bs1)
    alphaT = _sc_alpha(qG1, k1, src_pad, dst_pad, eaR)
    m = _segmax_combine(_sc_segmax(alphaT, dst_pad))
    d0 = _sc_wv(v01, m, alphaT, src_pad, dst2, eaR, 0)
    d1 = _sc_wv(v11, m, alphaT, src_pad, dst2, eaR, 1)
    qG2, k2, v02, v12, skip2 = _nodeout_proj(d0, d1, skip1, _weblk(We1).T,
                                             Wq2, bq2, Wk2, bk2, Wv2, bv2,
                                             We2, Ws2, bs2)

    alphaT2 = _sc_alpha(qG2, k2, src_pad, dst_pad, eaR)
    m2 = _segmax_combine(_sc_segmax(alphaT2, dst_pad))
    e0_ = _sc_wv(v02, m2, alphaT2, src_pad, dst2, eaR, 0)
    e1_ = _sc_wv(v12, m2, alphaT2, src_pad, dst2, eaR, 1)
    A = _nodeout_final(e0_, e1_, skip2, _weblk(We2).T)

    A = A[:NN].reshape(B_, N_, -1)
    aid = agent_id.astype(jnp.int32)[:, 0]
    return A[jnp.arange(B_), aid]
